# Initial kernel scaffold; baseline (speedup 1.0000x reference)
import jax
import jax.numpy as jnp
from jax.experimental import pallas as pl


def kernel(boxes, scores):
    raise NotImplementedError("write your pallas kernel here")


# speedup vs baseline: 28.3688x; 28.3688x over previous
import jax
import jax.numpy as jnp
from jax.experimental import pallas as pl

_IMG = 512.0
_IOU_T = 0.5
_SCORE_T = 0.4
_MAX_OUT = 100
_LANE = 128


def _nms_body(y0_ref, x0_ref, y1_ref, x1_ref, s_ref,
              oy0_ref, ox0_ref, oy1_ref, ox1_ref, osc_ref):
    y0 = y0_ref[...]
    x0 = x0_ref[...]
    y1 = y1_ref[...]
    x1 = x1_ref[...]
    s = s_ref[...]
    b, npad = s.shape

    h = y1 - y0
    w = x1 - x0
    area = h * w
    valid = (w / _IMG <= 1.0) & (h / _IMG <= 1.0) & (area > 100.0) & (s >= _SCORE_T)
    m0 = jnp.where(valid, s, -1.0)
    a2 = (y1 - y0) * (x1 - x0)

    idx_n = jax.lax.broadcasted_iota(jnp.int32, (b, npad), 1)
    iota_out = jax.lax.broadcasted_iota(jnp.int32, (b, _LANE), 1)

    def step(k, carry):
        m, ay0, ax0, ay1, ax1, asc = carry
        best = jnp.max(m, axis=1, keepdims=True)
        eq = m == best
        idx = jnp.min(jnp.where(eq, idx_n, npad), axis=1, keepdims=True)
        sel = idx_n == idx
        by0 = jnp.sum(jnp.where(sel, y0, 0.0), axis=1, keepdims=True)
        bx0 = jnp.sum(jnp.where(sel, x0, 0.0), axis=1, keepdims=True)
        by1 = jnp.sum(jnp.where(sel, y1, 0.0), axis=1, keepdims=True)
        bx1 = jnp.sum(jnp.where(sel, x1, 0.0), axis=1, keepdims=True)
        yA = jnp.maximum(by0, y0)
        xA = jnp.maximum(bx0, x0)
        yB = jnp.minimum(by1, y1)
        xB = jnp.minimum(bx1, x1)
        inter = jnp.maximum(yB - yA, 0.0) * jnp.maximum(xB - xA, 0.0)
        a1 = (by1 - by0) * (bx1 - bx0)
        iou = inter / (a1 + a2 - inter + 1e-8)
        supp = (iou > _IOU_T) | sel
        m2 = jnp.where(supp, -1.0, m)

        keep = best > 0.0
        at = (iota_out == k) & keep
        ay0 = jnp.where(at, by0, ay0)
        ax0 = jnp.where(at, bx0, ax0)
        ay1 = jnp.where(at, by1, ay1)
        ax1 = jnp.where(at, bx1, ax1)
        asc = jnp.where(at, best, asc)
        return m2, ay0, ax0, ay1, ax1, asc

    zero = jnp.zeros((b, _LANE), jnp.float32)
    carry = (m0, zero, zero, zero, zero, zero)
    _, ay0, ax0, ay1, ax1, asc = jax.lax.fori_loop(0, _MAX_OUT, step, carry)

    oy0_ref[...] = jnp.clip(ay0, 0.0, _IMG)
    ox0_ref[...] = jnp.clip(ax0, 0.0, _IMG)
    oy1_ref[...] = jnp.clip(ay1, 0.0, _IMG)
    ox1_ref[...] = jnp.clip(ax1, 0.0, _IMG)
    osc_ref[...] = asc


def kernel(boxes, scores):
    b, n, _ = boxes.shape
    npad = ((n + _LANE - 1) // _LANE) * _LANE
    pad = npad - n

    y0 = jnp.pad(boxes[..., 0], ((0, 0), (0, pad)))
    x0 = jnp.pad(boxes[..., 1], ((0, 0), (0, pad)))
    y1 = jnp.pad(boxes[..., 2], ((0, 0), (0, pad)))
    x1 = jnp.pad(boxes[..., 3], ((0, 0), (0, pad)))
    s = jnp.pad(scores, ((0, 0), (0, pad)), constant_values=-1.0)

    out_sd = [jax.ShapeDtypeStruct((b, _LANE), jnp.float32)] * 5
    oy0, ox0, oy1, ox1, osc = pl.pallas_call(
        _nms_body,
        out_shape=out_sd,
    )(y0, x0, y1, x1, s)

    sel_boxes = jnp.stack([oy0, ox0, oy1, ox1], axis=-1)[:, :_MAX_OUT, :]
    sel_scores = osc[:, :_MAX_OUT]
    max_scores = jnp.maximum(jnp.max(sel_scores, axis=1), 0.0)
    loss = jnp.sum(max_scores ** 2.0)
    return sel_boxes, sel_scores, loss


# speedup vs baseline: 54.7760x; 1.9309x over previous
import jax
import jax.numpy as jnp
from jax import lax
from jax.experimental import pallas as pl
from jax.experimental.pallas import tpu as pltpu
from jax.experimental.pallas import tpu_sc as plsc

_IMG = 512.0
_IOU_T = 0.5
_SCORE_T = 0.4
_MAX_OUT = 100
_L = 16
_OUTW = 112
_NCORES = 2
_NSUB = 16
_CHUNK = 64
_NEG = -3.0


def _fullf(v):
    return jnp.full((_L,), v, jnp.float32)


def _fulli(v):
    return jnp.full((_L,), v, jnp.int32)


def _hmax(x):
    m = x[0]
    for t in range(1, _L):
        m = jnp.maximum(m, x[t])
    return m


def _hmin(x):
    m = x[0]
    for t in range(1, _L):
        m = jnp.minimum(m, x[t])
    return m


def _sc_body(y0h, x0h, y1h, x1h, sh,
             oy0h, ox0h, oy1h, ox1h, osch,
             y0v, x0v, y1v, x1v, mv,
             l1v, l1i, l2v, l2i,
             ky0, kx0, ky1, kx1, ka, ksc,
             qv, sm):
    b_total, n = sh.shape
    nv = n // _L
    g1 = (nv + _L - 1) // _L
    g1p = ((g1 + _L - 1) // _L) * _L
    g2 = g1p // _L
    max_pops = n // _CHUNK + 2

    cid = lax.axis_index("c")
    sid = lax.axis_index("s")
    wid = sid * _NCORES + cid
    lane = lax.iota(jnp.int32, _L)

    @pl.when(wid < b_total)
    def _():
        b = wid
        pltpu.sync_copy(y0h.at[b], y0v)
        pltpu.sync_copy(x0h.at[b], x0v)
        pltpu.sync_copy(y1h.at[b], y1v)
        pltpu.sync_copy(x1h.at[b], x1v)
        pltpu.sync_copy(sh.at[b], mv)

        for j in range(_OUTW // _L):
            sl = pl.ds(j * _L, _L)
            ky0[sl] = _fullf(2e9)
            kx0[sl] = _fullf(2e9)
            ky1[sl] = _fullf(1e9)
            kx1[sl] = _fullf(1e9)
            ka[sl] = _fullf(0.0)
            ksc[sl] = _fullf(0.0)

        def build_g(g, c):
            def build_j(j, acc):
                accv, acci = acc
                off = (g * _L + j) * _L
                s = mv[pl.ds(off, _L)]
                a0 = y0v[pl.ds(off, _L)]
                b0 = x0v[pl.ds(off, _L)]
                a1 = y1v[pl.ds(off, _L)]
                b1 = x1v[pl.ds(off, _L)]
                h = a1 - a0
                w = b1 - b0
                area = h * w
                valid = ((w / _IMG <= 1.0) & (h / _IMG <= 1.0)
                         & (area > 100.0) & (s >= _SCORE_T))
                m = jnp.where(valid, s, -1.0)
                mv[pl.ds(off, _L)] = m
                gt = m > accv
                accv = jnp.where(gt, m, accv)
                acci = jnp.where(gt, off + lane, acci)
                return accv, acci

            nj = jnp.maximum(jnp.minimum(_L, nv - g * _L), 0)
            accv, acci = lax.fori_loop(0, nj, build_j,
                                       (_fullf(_NEG), _fulli(0)))
            l1v[pl.ds(g * _L, _L)] = accv
            l1i[pl.ds(g * _L, _L)] = acci
            return c

        lax.fori_loop(0, g1p, build_g, jnp.int32(0))

        def build_h(h, c):
            def bj(j, acc):
                accv, acci = acc
                t = h * _L + j
                v1 = l1v[pl.ds(t * _L, _L)]
                i1 = l1i[pl.ds(t * _L, _L)]
                gt = v1 > accv
                return jnp.where(gt, v1, accv), jnp.where(gt, i1, acci)

            accv, acci = lax.fori_loop(0, _L, bj, (_fullf(_NEG), _fulli(0)))
            l2v[pl.ds(h * _L, _L)] = accv
            l2i[pl.ds(h * _L, _L)] = acci
            return c

        lax.fori_loop(0, g2, build_h, jnp.int32(0))

        sm[0] = jnp.int32(0)
        sm[1] = jnp.int32(0)

        def phase_body(p, c):
            nb = jnp.where(sm[1] == 1, 0, _CHUNK)

            def pop_body(t, c2):
                @pl.when(sm[1] == 0)
                def _():
                    def rj(j, acc):
                        accv, acci = acc
                        v2 = l2v[pl.ds(j * _L, _L)]
                        i2 = l2i[pl.ds(j * _L, _L)]
                        gt = v2 > accv
                        return (jnp.where(gt, v2, accv),
                                jnp.where(gt, i2, acci))

                    rv, ri = lax.fori_loop(0, g2, rj,
                                           (_fullf(_NEG), _fulli(0)))
                    mx = _hmax(rv)
                    pos = _hmin(jnp.where(rv == mx, ri, 2**30))
                    pos = jnp.minimum(pos, n - _L)
                    exhausted = mx < _SCORE_T

                    cy0 = y0v[pl.ds(pos, _L)][0]
                    cx0 = x0v[pl.ds(pos, _L)][0]
                    cy1 = y1v[pl.ds(pos, _L)][0]
                    cx1 = x1v[pl.ds(pos, _L)][0]
                    ca = (cy1 - cy0) * (cx1 - cx0)

                    mi = _fullf(0.0)
                    for j in range(_OUTW // _L):
                        sl = pl.ds(j * _L, _L)
                        ya = jnp.maximum(ky0[sl], cy0)
                        xa = jnp.maximum(kx0[sl], cx0)
                        yb = jnp.minimum(ky1[sl], cy1)
                        xb = jnp.minimum(kx1[sl], cx1)
                        inter = (jnp.maximum(yb - ya, 0.0)
                                 * jnp.maximum(xb - xa, 0.0))
                        iou = inter / (ka[sl] + ca - inter + 1e-8)
                        mi = jnp.maximum(mi, iou)
                    keep = ((_hmax(mi) <= _IOU_T)
                            & jnp.logical_not(exhausted))

                    base = (pos // _L) * _L
                    lpos = pos - base
                    old = mv[pl.ds(base, _L)]
                    ctgt = jnp.where(exhausted, -1, lpos)
                    mv[pl.ds(base, _L)] = jnp.where(lane == ctgt, -1.0, old)

                    g = pos // (_L * _L)

                    def rb_j(j, acc):
                        accv, acci = acc
                        off = (g * _L + j) * _L
                        m = mv[pl.ds(off, _L)]
                        gt = m > accv
                        return (jnp.where(gt, m, accv),
                                jnp.where(gt, off + lane, acci))

                    nj = jnp.maximum(jnp.minimum(_L, nv - g * _L), 0)
                    accv, acci = lax.fori_loop(0, nj, rb_j,
                                               (_fullf(_NEG), _fulli(0)))
                    l1v[pl.ds(g * _L, _L)] = accv
                    l1i[pl.ds(g * _L, _L)] = acci

                    h = g // _L

                    def rb2_j(j, acc):
                        accv, acci = acc
                        t = h * _L + j
                        v1 = l1v[pl.ds(t * _L, _L)]
                        i1 = l1i[pl.ds(t * _L, _L)]
                        gt = v1 > accv
                        return (jnp.where(gt, v1, accv),
                                jnp.where(gt, i1, acci))

                    accv2, acci2 = lax.fori_loop(0, _L, rb2_j,
                                                 (_fullf(_NEG), _fulli(0)))
                    l2v[pl.ds(h * _L, _L)] = accv2
                    l2i[pl.ds(h * _L, _L)] = acci2

                    kn = sm[0]
                    kb = (kn // _L) * _L
                    atgt = jnp.where(keep, kn - kb, -1)

                    def rmw(ref, val):
                        sl2 = pl.ds(kb, _L)
                        ref[sl2] = jnp.where(lane == atgt, val, ref[sl2])

                    rmw(ky0, cy0)
                    rmw(kx0, cx0)
                    rmw(ky1, cy1)
                    rmw(kx1, cx1)
                    rmw(ka, ca)
                    rmw(ksc, mx)

                    kn2 = kn + keep.astype(jnp.int32)
                    sm[0] = kn2
                    sm[1] = jnp.where(exhausted | (kn2 >= _MAX_OUT),
                                      1, 0).astype(jnp.int32)
                return c2

            lax.fori_loop(0, nb, pop_body, c)
            return c

        lax.fori_loop(0, max_pops, phase_body, jnp.int32(0))

        kn_f = sm[0]
        for arr, oref in ((ky0, oy0h), (kx0, ox0h), (ky1, oy1h),
                          (kx1, ox1h), (ksc, osch)):
            for j in range(_OUTW // _L):
                sl = pl.ds(j * _L, _L)
                slot = lane + j * _L
                qv[sl] = jnp.where(slot < kn_f, arr[sl], 0.0)
            pltpu.sync_copy(qv, oref.at[b])


def _make_sc_call(b, n):
    f32 = jnp.float32
    i32 = jnp.int32
    nv = n // _L
    g1p = ((((nv + _L - 1) // _L) + _L - 1) // _L) * _L
    mesh = plsc.VectorSubcoreMesh(core_axis_name="c", subcore_axis_name="s",
                                  num_cores=_NCORES, num_subcores=_NSUB)
    out_type = [jax.ShapeDtypeStruct((b, _OUTW), f32)] * 5
    scratch = (
        [pltpu.VMEM((n,), f32)] * 5
        + [pltpu.VMEM((g1p * _L,), f32), pltpu.VMEM((g1p * _L,), i32),
           pltpu.VMEM((g1p,), f32), pltpu.VMEM((g1p,), i32)]
        + [pltpu.VMEM((_OUTW,), f32)] * 6
        + [pltpu.VMEM((_OUTW,), f32)]
        + [pltpu.SMEM((4,), i32)]
    )
    return pl.kernel(_sc_body, out_type, mesh=mesh, scratch_types=scratch)


def kernel(boxes, scores):
    b, n, _ = boxes.shape
    npad = ((n + _L - 1) // _L) * _L + _L
    pad = npad - n
    y0 = jnp.pad(boxes[..., 0], ((0, 0), (0, pad)))
    x0 = jnp.pad(boxes[..., 1], ((0, 0), (0, pad)))
    y1 = jnp.pad(boxes[..., 2], ((0, 0), (0, pad)))
    x1 = jnp.pad(boxes[..., 3], ((0, 0), (0, pad)))
    s = jnp.pad(scores, ((0, 0), (0, pad)))

    oy0, ox0, oy1, ox1, osc = _make_sc_call(b, npad)(y0, x0, y1, x1, s)

    sel_boxes = jnp.clip(
        jnp.stack([oy0, ox0, oy1, ox1], axis=-1)[:, :_MAX_OUT, :], 0.0, _IMG)
    sel_scores = osc[:, :_MAX_OUT]
    max_scores = jnp.maximum(jnp.max(sel_scores, axis=1), 0.0)
    loss = jnp.sum(max_scores ** 2.0)
    return sel_boxes, sel_scores, loss
